# baseline (speedup 1.0000x reference)
import functools

import jax
import jax.numpy as jnp
from jax import lax
from jax.experimental import pallas as pl
from jax.experimental.pallas import tpu as pltpu
from jax.experimental.pallas import tpu_sc as plsc

_CHUNK = 128


def _sc_gather(users, movies, user_table, movie_table):
    B = users.shape[0]
    D = user_table.shape[1]
    info = plsc.get_sparse_core_info()
    nc, ns = info.num_cores, info.num_subcores
    nw = nc * ns
    b_per_w = B // nw
    n_ch = b_per_w // _CHUNK

    mesh = plsc.VectorSubcoreMesh(core_axis_name="c", subcore_axis_name="s")

    @functools.partial(
        pl.kernel,
        mesh=mesh,
        out_type=(
            jax.ShapeDtypeStruct((B, D), jnp.float32),
            jax.ShapeDtypeStruct((B, D), jnp.float32),
        ),
        scratch_types=[
            pltpu.VMEM((n_ch, _CHUNK), jnp.int32),
            pltpu.VMEM((n_ch, _CHUNK), jnp.int32),
            pltpu.VMEM((b_per_w, D), jnp.float32),
            pltpu.VMEM((b_per_w, D), jnp.float32),
            pltpu.SemaphoreType.DMA,
        ],
        compiler_params=pltpu.CompilerParams(use_tc_tiling_on_sc=False),
    )
    def gather_kernel(users_hbm, movies_hbm, utab_hbm, mtab_hbm,
                      uout_hbm, mout_hbm,
                      uidx_v, midx_v, urows_v, mrows_v, sem):
        wid = lax.axis_index("s") * nc + lax.axis_index("c")
        base = wid * b_per_w
        for j in range(n_ch):
            pltpu.sync_copy(users_hbm.at[pl.ds(base + j * _CHUNK, _CHUNK)],
                            uidx_v.at[j])
            pltpu.sync_copy(movies_hbm.at[pl.ds(base + j * _CHUNK, _CHUNK)],
                            midx_v.at[j])
        copies = []
        for j in range(n_ch):
            copies.append(pltpu.async_copy(
                utab_hbm.at[uidx_v.at[j]],
                urows_v.at[pl.ds(j * _CHUNK, _CHUNK)], sem))
            copies.append(pltpu.async_copy(
                mtab_hbm.at[midx_v.at[j]],
                mrows_v.at[pl.ds(j * _CHUNK, _CHUNK)], sem))
        for c in copies:
            c.wait()
        pltpu.sync_copy(urows_v, uout_hbm.at[pl.ds(base, b_per_w)])
        pltpu.sync_copy(mrows_v, mout_hbm.at[pl.ds(base, b_per_w)])

    return gather_kernel(users, movies, user_table, movie_table)


def _mlp_body(u_ref, m_ref, w1u_ref, w1m_ref, b1_ref, w2_ref, b2_ref, out_ref):
    h = jnp.dot(u_ref[...], w1u_ref[...], preferred_element_type=jnp.float32)
    h = h + jnp.dot(m_ref[...], w1m_ref[...], preferred_element_type=jnp.float32)
    h = jnp.maximum(h + b1_ref[...], 0.0)
    out_ref[...] = (
        jnp.dot(h, w2_ref[...], preferred_element_type=jnp.float32)
        + b2_ref[...])


def _tc_mlp(u_emb, m_emb, W1, b1, W2, b2, block_b=2048):
    B, D = u_emb.shape
    H = W1.shape[1]
    W1u = W1[:D]
    W1m = W1[D:]
    b1r = b1.reshape(1, H)
    b2r = b2.reshape(1, 1)
    grid = (B // block_b,)
    return pl.pallas_call(
        _mlp_body,
        grid=grid,
        in_specs=[
            pl.BlockSpec((block_b, D), lambda i: (i, 0)),
            pl.BlockSpec((block_b, D), lambda i: (i, 0)),
            pl.BlockSpec((D, H), lambda i: (0, 0)),
            pl.BlockSpec((D, H), lambda i: (0, 0)),
            pl.BlockSpec((1, H), lambda i: (0, 0)),
            pl.BlockSpec((H, 1), lambda i: (0, 0)),
            pl.BlockSpec((1, 1), lambda i: (0, 0)),
        ],
        out_specs=pl.BlockSpec((block_b, 1), lambda i: (i, 0)),
        out_shape=jax.ShapeDtypeStruct((B, 1), jnp.float32),
    )(u_emb, m_emb, W1u, W1m, b1r, W2, b2r)


def kernel(users, movies, user_table, movie_table, W1, b1, W2, b2):
    users = users.astype(jnp.int32)
    movies = movies.astype(jnp.int32)
    u_emb, m_emb = _sc_gather(users, movies, user_table, movie_table)
    return _tc_mlp(u_emb, m_emb, W1, b1, W2, b2)


# speedup vs baseline: 1.0063x; 1.0063x over previous
import functools

import jax
import jax.numpy as jnp
from jax import lax
from jax.experimental import pallas as pl
from jax.experimental.pallas import tpu as pltpu
from jax.experimental.pallas import tpu_sc as plsc

_CHUNK = 128


def _sc_gather_pairs(uidx2, midx2, utab2, mtab2):
    B = uidx2.shape[0]
    D2 = utab2.shape[1]
    info = plsc.get_sparse_core_info()
    nc, ns = info.num_cores, info.num_subcores
    nw = nc * ns
    b_per_w = B // nw
    n_ch = b_per_w // _CHUNK

    mesh = plsc.VectorSubcoreMesh(core_axis_name="c", subcore_axis_name="s")

    @functools.partial(
        pl.kernel,
        mesh=mesh,
        out_type=(
            jax.ShapeDtypeStruct((B, D2), jnp.float32),
            jax.ShapeDtypeStruct((B, D2), jnp.float32),
        ),
        scratch_types=[
            pltpu.VMEM((n_ch, _CHUNK), jnp.int32),
            pltpu.VMEM((n_ch, _CHUNK), jnp.int32),
            pltpu.VMEM((b_per_w, D2), jnp.float32),
            pltpu.SemaphoreType.DMA,
        ],
    )
    def gather_kernel(uidx_hbm, midx_hbm, utab_hbm, mtab_hbm,
                      uout_hbm, mout_hbm,
                      uidx_v, midx_v, rows_v, sem):
        wid = lax.axis_index("s") * nc + lax.axis_index("c")
        base = wid * b_per_w
        for j in range(n_ch):
            pltpu.sync_copy(uidx_hbm.at[pl.ds(base + j * _CHUNK, _CHUNK)],
                            uidx_v.at[j])
            pltpu.sync_copy(midx_hbm.at[pl.ds(base + j * _CHUNK, _CHUNK)],
                            midx_v.at[j])
        copies = []
        for j in range(n_ch):
            copies.append(pltpu.async_copy(
                utab_hbm.at[uidx_v.at[j]],
                rows_v.at[pl.ds(j * _CHUNK, _CHUNK)], sem))
        for c in copies:
            c.wait()
        pltpu.sync_copy(rows_v, uout_hbm.at[pl.ds(base, b_per_w)])
        copies = []
        for j in range(n_ch):
            copies.append(pltpu.async_copy(
                mtab_hbm.at[midx_v.at[j]],
                rows_v.at[pl.ds(j * _CHUNK, _CHUNK)], sem))
        for c in copies:
            c.wait()
        pltpu.sync_copy(rows_v, mout_hbm.at[pl.ds(base, b_per_w)])

    return gather_kernel(uidx2, midx2, utab2, mtab2)


def _mlp_body(u2_ref, m2_ref, pu_ref, pm_ref, w1u_ref, w1m_ref, b1_ref,
              w2_ref, b2_ref, out_ref):
    D = w1u_ref.shape[0] // 2
    col = lax.broadcasted_iota(jnp.int32, u2_ref.shape, 1)
    lo = (col < D).astype(jnp.float32)
    hi = 1.0 - lo
    mask_u = lo * (1.0 - pu_ref[...]) + hi * pu_ref[...]
    mask_m = lo * (1.0 - pm_ref[...]) + hi * pm_ref[...]
    h = jnp.dot(u2_ref[...] * mask_u, w1u_ref[...],
                preferred_element_type=jnp.float32)
    h = h + jnp.dot(m2_ref[...] * mask_m, w1m_ref[...],
                    preferred_element_type=jnp.float32)
    h = jnp.maximum(h + b1_ref[...], 0.0)
    out_ref[...] = (
        jnp.dot(h, w2_ref[...], preferred_element_type=jnp.float32)
        + b2_ref[...])


def _tc_mlp(u2, m2, pu, pm, W1, b1, W2, b2, block_b=2048):
    B, D2 = u2.shape
    D = D2 // 2
    H = W1.shape[1]
    W1u2 = jnp.concatenate([W1[:D], W1[:D]], axis=0)
    W1m2 = jnp.concatenate([W1[D:], W1[D:]], axis=0)
    b1r = b1.reshape(1, H)
    b2r = b2.reshape(1, 1)
    grid = (B // block_b,)
    return pl.pallas_call(
        _mlp_body,
        grid=grid,
        in_specs=[
            pl.BlockSpec((block_b, D2), lambda i: (i, 0)),
            pl.BlockSpec((block_b, D2), lambda i: (i, 0)),
            pl.BlockSpec((block_b, 1), lambda i: (i, 0)),
            pl.BlockSpec((block_b, 1), lambda i: (i, 0)),
            pl.BlockSpec((D2, H), lambda i: (0, 0)),
            pl.BlockSpec((D2, H), lambda i: (0, 0)),
            pl.BlockSpec((1, H), lambda i: (0, 0)),
            pl.BlockSpec((H, 1), lambda i: (0, 0)),
            pl.BlockSpec((1, 1), lambda i: (0, 0)),
        ],
        out_specs=pl.BlockSpec((block_b, 1), lambda i: (i, 0)),
        out_shape=jax.ShapeDtypeStruct((B, 1), jnp.float32),
    )(u2, m2, pu, pm, W1u2, W1m2, b1r, W2, b2r)


def kernel(users, movies, user_table, movie_table, W1, b1, W2, b2):
    users = users.astype(jnp.int32)
    movies = movies.astype(jnp.int32)
    V, D = user_table.shape
    utab2 = user_table.reshape(V // 2, 2 * D)
    mtab2 = movie_table.reshape(V // 2, 2 * D)
    pu = (users & 1).astype(jnp.float32).reshape(-1, 1)
    pm = (movies & 1).astype(jnp.float32).reshape(-1, 1)
    u2, m2 = _sc_gather_pairs(users >> 1, movies >> 1, utab2, mtab2)
    return _tc_mlp(u2, m2, pu, pm, W1, b1, W2, b2)
